# baseline (device time: 197198 ns/iter reference)
import jax
import jax.numpy as jnp
from jax import lax
from jax.experimental import pallas as pl
from jax.experimental.pallas import tpu as pltpu

N_DEV = 4


def kernel(x, dest):
    m, n = x.shape
    d2 = dest.reshape(16, 128)
    dm, dn = d2.shape

    def body(x_ref, d_ref, xout_ref, dout_ref,
             send_x, recv_x, send_d, recv_d):
        my_x = lax.axis_index("x")
        my_y = lax.axis_index("y")
        my_z = lax.axis_index("z")
        left = lax.rem(my_z - 1 + N_DEV, N_DEV)
        right = lax.rem(my_z + 1, N_DEV)

        barrier_sem = pltpu.get_barrier_semaphore()
        for nbr in [left, right]:
            pl.semaphore_signal(
                barrier_sem, inc=1,
                device_id=(my_x, my_y, nbr),
                device_id_type=pl.DeviceIdType.MESH,
            )
        pl.semaphore_wait(barrier_sem, 2)

        xout_ref[pl.ds(my_z * m, m), :] = x_ref[...].astype(jnp.bfloat16)
        dout_ref[pl.ds(my_z * dm, dm), :] = d_ref[...]

        for h in range(N_DEV - 1):
            origin = lax.rem(my_z - h + N_DEV, N_DEV)
            rx = pltpu.make_async_remote_copy(
                src_ref=xout_ref.at[pl.ds(origin * m, m), :],
                dst_ref=xout_ref.at[pl.ds(origin * m, m), :],
                send_sem=send_x.at[h],
                recv_sem=recv_x.at[h],
                device_id=(my_x, my_y, right),
                device_id_type=pl.DeviceIdType.MESH,
            )
            rd = pltpu.make_async_remote_copy(
                src_ref=dout_ref.at[pl.ds(origin * dm, dm), :],
                dst_ref=dout_ref.at[pl.ds(origin * dm, dm), :],
                send_sem=send_d.at[h],
                recv_sem=recv_d.at[h],
                device_id=(my_x, my_y, right),
                device_id_type=pl.DeviceIdType.MESH,
            )
            rx.start()
            rd.start()
            rx.wait()
            rd.wait()

    x_full, d_full = pl.pallas_call(
        body,
        out_shape=(
            jax.ShapeDtypeStruct((N_DEV * m, n), jnp.bfloat16),
            jax.ShapeDtypeStruct((N_DEV * dm, dn), jnp.int32),
        ),
        in_specs=[
            pl.BlockSpec(memory_space=pltpu.VMEM),
            pl.BlockSpec(memory_space=pltpu.VMEM),
        ],
        out_specs=(
            pl.BlockSpec(memory_space=pltpu.VMEM),
            pl.BlockSpec(memory_space=pltpu.VMEM),
        ),
        scratch_shapes=[
            pltpu.SemaphoreType.DMA((N_DEV - 1,)),
            pltpu.SemaphoreType.DMA((N_DEV - 1,)),
            pltpu.SemaphoreType.DMA((N_DEV - 1,)),
            pltpu.SemaphoreType.DMA((N_DEV - 1,)),
        ],
        compiler_params=pltpu.CompilerParams(collective_id=0),
    )(x, d2)

    dest_full = d_full.reshape(N_DEV * m)
    order = jnp.argsort(dest_full, stable=True)
    my_z = lax.axis_index("z")
    idx = lax.dynamic_slice(order, (my_z * m,), (m,))
    return jnp.take(x_full, idx, axis=0)
